# baseline (speedup 1.0000x reference)
import functools
import math

import jax
import jax.numpy as jnp
from jax.experimental import pallas as pl
from jax.experimental.pallas import tpu as pltpu
from jax.experimental.pallas import tpu_sc as plsc

_SCALE = 64.0
_MARGIN = 0.5
_COS_M = math.cos(_MARGIN)
_SIN_M = math.sin(_MARGIN)
_LOG2E = math.log2(math.e)
_SE = _SCALE * _LOG2E

_R = 32
_CW = 2048

_SC_ROWS = 512
_SC_CW = 1024
_SC_NJ = 97
_CR = 64


def _sc_gather_rows(logits, labels):
    n_rows, n_cols = logits.shape

    @functools.partial(
        pl.kernel,
        out_type=jax.ShapeDtypeStruct((n_rows, 128), logits.dtype),
        mesh=plsc.ScalarSubcoreMesh(axis_name="c", num_cores=2),
        scratch_types=[
            pltpu.SMEM((n_rows,), jnp.int32),
            pltpu.SemaphoreType.DMA,
            pltpu.SemaphoreType.DMA,
        ],
    )
    def gather_kernel(x_hbm, l_hbm, o_hbm, l_smem, sem_l, sem_d):
        core = jax.lax.axis_index("c")
        pltpu.async_copy(l_hbm, l_smem, sem_l).wait()
        half = n_rows // 2
        base = core * half

        @pl.loop(0, half)
        def _(i):
            r = base + i
            st = (l_smem[r] // 128) * 128
            pltpu.async_copy(x_hbm.at[r, pl.ds(st, 128)], o_hbm.at[r], sem_d)

        @pl.loop(0, half)
        def _(i):
            r = base + i
            st = (l_smem[r] // 128) * 128
            pltpu.make_async_copy(
                x_hbm.at[r, pl.ds(st, 128)], o_hbm.at[r], sem_d
            ).wait()

    return gather_kernel(logits, labels)


def _sc_sumexp(logits):
    n_rows, n_cols = logits.shape
    row_blk0 = (n_rows - _SC_ROWS) // 8

    @functools.partial(
        pl.kernel,
        out_type=jax.ShapeDtypeStruct((_SC_ROWS, _SC_NJ * 128), jnp.float32),
        mesh=plsc.VectorSubcoreMesh(core_axis_name="c", subcore_axis_name="s"),
    )
    def sumexp_kernel(x_hbm, o_hbm):
        def body(in_vmem, out_vmem):
            @pl.loop(0, 8)
            def _(r):
                @pl.loop(0, 8)
                def _(k):
                    sl_o = (pl.ds(r, 1), pl.ds(k * 16, 16))
                    xc = jnp.clip(
                        in_vmem.at[pl.ds(r, 1), pl.ds(k * 16, 16)][...],
                        -1.0, 1.0,
                    )
                    out_vmem.at[*sl_o][...] = jnp.exp(xc * _SCALE - _SCALE)
                    for m in range(1, 8):
                        p = m * 8 + k
                        xc = jnp.clip(
                            in_vmem.at[pl.ds(r, 1), pl.ds(p * 16, 16)][...],
                            -1.0, 1.0,
                        )
                        out_vmem.at[*sl_o][...] += jnp.exp(xc * _SCALE - _SCALE)

        pltpu.emit_pipeline(
            body,
            grid=(_SC_ROWS // 8, _SC_NJ),
            in_specs=[
                pl.BlockSpec((8, _SC_CW), index_map=lambda i, j: (row_blk0 + i, j))
            ],
            out_specs=[pl.BlockSpec((8, 128), index_map=lambda i, j: (i, j))],
            core_axis_name=("c", "s"),
            dimension_semantics=(pltpu.PARALLEL, pltpu.PARALLEL),
        )(x_hbm, o_hbm)

    return sumexp_kernel(logits)


def _tree128(v):
    parts = [v[:, k * 128:(k + 1) * 128] for k in range(v.shape[1] // 128)]
    while len(parts) > 1:
        half = (len(parts) + 1) // 2
        parts = [
            parts[m] + parts[m + half] if m + half < len(parts) else parts[m]
            for m in range(half)
        ]
    return parts[0]


def _margin_row_loss(s0, c, n_rows):
    t_new = _SCALE * (c * _COS_M - _SIN_M * jnp.sqrt(jnp.maximum(1.0 - c * c, 0.0)))
    e_old = jnp.exp2(c * _SE - _SE)
    e_new = jnp.exp(t_new - _SCALE)
    s = s0 - e_old + e_new
    return (_SCALE + jnp.log(s) - t_new) * (1.0 / n_rows)


def _pick_label(lane_ref, x128_ref, rows):
    onehot = jax.lax.broadcasted_iota(jnp.int32, (rows, 128), 1) == lane_ref[...]
    return jnp.sum(jnp.where(onehot, jnp.clip(x128_ref[...], -1.0, 1.0), 0.0), axis=1)


def _loss_body(lane_ref, x128_ref, x_ref, out_ref, *, n_rows, n_cols):
    i = pl.program_id(0)

    n_full = n_cols // _CW
    tail = n_cols - n_full * _CW

    def col_body(j, acc):
        xc = jnp.clip(x_ref[:, pl.ds(j * _CW, _CW)], -1.0, 1.0)
        return acc + _tree128(jnp.exp2(xc * _SE - _SE))

    acc = jax.lax.fori_loop(
        0, n_full, col_body, jnp.zeros((_R, 128), jnp.float32), unroll=4
    )
    s0 = jnp.sum(acc, axis=1)
    if tail:
        xc = jnp.clip(x_ref[:, pl.ds(n_full * _CW, tail)], -1.0, 1.0)
        s0 = s0 + jnp.sum(jnp.exp2(xc * _SE - _SE), axis=1)

    c = _pick_label(lane_ref, x128_ref, _R)
    row_loss = _margin_row_loss(s0, c, n_rows)

    @pl.when(i == 0)
    def _():
        out_ref[0, 0] = 0.0

    out_ref[0, 0] += jnp.sum(row_loss)


def _combine_body(a_ref, lane_ref, x128_ref, part_ref, xt_ref, out_ref, *,
                  n_rows, n_cols):
    i = pl.program_id(0)

    def cb(j, acc):
        return acc + part_ref[:, pl.ds(j * 128, 128)]

    acc = jax.lax.fori_loop(
        0, _SC_NJ, cb, jnp.zeros((_CR, 128), jnp.float32), unroll=4
    )
    s0 = jnp.sum(acc, axis=1)

    tail0 = _SC_NJ * _SC_CW
    valid = jax.lax.broadcasted_iota(jnp.int32, (_CR, _SC_CW), 1) < (n_cols - tail0)
    xc = jnp.clip(xt_ref[...], -1.0, 1.0)
    e = jnp.where(valid, jnp.exp2(xc * _SE - _SE), 0.0)
    s0 = s0 + jnp.sum(e, axis=1)

    c = _pick_label(lane_ref, x128_ref, _CR)
    row_loss = _margin_row_loss(s0, c, n_rows)

    @pl.when(i == 0)
    def _():
        out_ref[0, 0] = a_ref[0, 0]

    out_ref[0, 0] += jnp.sum(row_loss)


@jax.jit
def kernel(logits, labels):
    n_rows, n_cols = logits.shape
    labels = labels.astype(jnp.int32)
    tc_rows = n_rows - _SC_ROWS

    x128 = _sc_gather_rows(logits, labels)
    lane128 = (labels % 128).reshape(n_rows, 1)

    part = _sc_sumexp(logits)

    a = pl.pallas_call(
        functools.partial(_loss_body, n_rows=n_rows, n_cols=n_cols),
        grid=(tc_rows // _R,),
        in_specs=[
            pl.BlockSpec((_R, 1), lambda i: (i, 0)),
            pl.BlockSpec((_R, 128), lambda i: (i, 0)),
            pl.BlockSpec((_R, n_cols), lambda i: (i, 0)),
        ],
        out_specs=pl.BlockSpec((1, 1), lambda i: (0, 0), memory_space=pltpu.SMEM),
        out_shape=jax.ShapeDtypeStruct((1, 1), jnp.float32),
    )(lane128, x128, logits)

    blk0 = tc_rows // _CR
    out = pl.pallas_call(
        functools.partial(_combine_body, n_rows=n_rows, n_cols=n_cols),
        grid=(_SC_ROWS // _CR,),
        in_specs=[
            pl.BlockSpec((1, 1), lambda i: (0, 0), memory_space=pltpu.SMEM),
            pl.BlockSpec((_CR, 1), lambda i: (blk0 + i, 0)),
            pl.BlockSpec((_CR, 128), lambda i: (blk0 + i, 0)),
            pl.BlockSpec((_CR, _SC_NJ * 128), lambda i: (i, 0)),
            pl.BlockSpec((_CR, _SC_CW), lambda i: (blk0 + i, _SC_NJ)),
        ],
        out_specs=pl.BlockSpec((1, 1), lambda i: (0, 0), memory_space=pltpu.SMEM),
        out_shape=jax.ShapeDtypeStruct((1, 1), jnp.float32),
    )(a, lane128, x128, part, logits)
    return out[0, 0]


# speedup vs baseline: 2.7293x; 2.7293x over previous
import functools
import math

import jax
import jax.numpy as jnp
from jax.experimental import pallas as pl
from jax.experimental.pallas import tpu as pltpu
from jax.experimental.pallas import tpu_sc as plsc

_SCALE = 64.0
_MARGIN = 0.5
_COS_M = math.cos(_MARGIN)
_SIN_M = math.sin(_MARGIN)
_LOG2E = math.log2(math.e)
_SE = _SCALE * _LOG2E

_R = 32
_CW = 2048

_SC_ROWS = 512
_SC_CW = 1024
_SC_NJ = 97
_CR = 64


def _sc_gather_rows(logits, labels):
    n_rows, n_cols = logits.shape

    @functools.partial(
        pl.kernel,
        out_type=jax.ShapeDtypeStruct((n_rows, 128), logits.dtype),
        mesh=plsc.ScalarSubcoreMesh(axis_name="c", num_cores=2),
        scratch_types=[
            pltpu.SMEM((n_rows,), jnp.int32),
            pltpu.SemaphoreType.DMA,
            pltpu.SemaphoreType.DMA,
        ],
    )
    def gather_kernel(x_hbm, l_hbm, o_hbm, l_smem, sem_l, sem_d):
        core = jax.lax.axis_index("c")
        pltpu.async_copy(l_hbm, l_smem, sem_l).wait()
        half = n_rows // 2
        base = core * half

        @pl.loop(0, half)
        def _(i):
            r = base + i
            st = (l_smem[r] // 128) * 128
            pltpu.async_copy(x_hbm.at[r, pl.ds(st, 128)], o_hbm.at[r], sem_d)

        @pl.loop(0, half)
        def _(i):
            r = base + i
            st = (l_smem[r] // 128) * 128
            pltpu.make_async_copy(
                x_hbm.at[r, pl.ds(st, 128)], o_hbm.at[r], sem_d
            ).wait()

    return gather_kernel(logits, labels)


def _sc_sumexp(logits):
    n_rows, n_cols = logits.shape
    row_blk0 = (n_rows - _SC_ROWS) // 8

    @functools.partial(
        pl.kernel,
        out_type=jax.ShapeDtypeStruct((_SC_ROWS, _SC_NJ * 128), jnp.float32),
        mesh=plsc.VectorSubcoreMesh(core_axis_name="c", subcore_axis_name="s"),
    )
    def sumexp_kernel(x_hbm, o_hbm):
        def body(in_vmem, out_vmem):
            @pl.loop(0, 8)
            def _(r):
                for k in range(8):
                    acc = None
                    for m in range(8):
                        p = m * 8 + k
                        xc = jnp.clip(
                            in_vmem.at[pl.ds(r, 1), pl.ds(p * 16, 16)][...],
                            -1.0, 1.0,
                        )
                        e = jnp.exp(xc * _SCALE - _SCALE)
                        acc = e if acc is None else acc + e
                    out_vmem.at[pl.ds(r, 1), pl.ds(k * 16, 16)][...] = acc

        pltpu.emit_pipeline(
            body,
            grid=(_SC_ROWS // 8, _SC_NJ),
            in_specs=[
                pl.BlockSpec((8, _SC_CW), index_map=lambda i, j: (row_blk0 + i, j))
            ],
            out_specs=[pl.BlockSpec((8, 128), index_map=lambda i, j: (i, j))],
            core_axis_name=("c", "s"),
            dimension_semantics=(pltpu.PARALLEL, pltpu.PARALLEL),
        )(x_hbm, o_hbm)

    return sumexp_kernel(logits)


def _tree128(v):
    parts = [v[:, k * 128:(k + 1) * 128] for k in range(v.shape[1] // 128)]
    while len(parts) > 1:
        half = (len(parts) + 1) // 2
        parts = [
            parts[m] + parts[m + half] if m + half < len(parts) else parts[m]
            for m in range(half)
        ]
    return parts[0]


def _margin_row_loss(s0, c, n_rows):
    t_new = _SCALE * (c * _COS_M - _SIN_M * jnp.sqrt(jnp.maximum(1.0 - c * c, 0.0)))
    e_old = jnp.exp2(c * _SE - _SE)
    e_new = jnp.exp(t_new - _SCALE)
    s = s0 - e_old + e_new
    return (_SCALE + jnp.log(s) - t_new) * (1.0 / n_rows)


def _pick_label(lane_ref, x128_ref, rows):
    onehot = jax.lax.broadcasted_iota(jnp.int32, (rows, 128), 1) == lane_ref[...]
    return jnp.sum(jnp.where(onehot, jnp.clip(x128_ref[...], -1.0, 1.0), 0.0), axis=1)


def _loss_body(lane_ref, x128_ref, x_ref, out_ref, *, n_rows, n_cols):
    i = pl.program_id(0)

    n_full = n_cols // _CW
    tail = n_cols - n_full * _CW

    def col_body(j, acc):
        xc = jnp.clip(x_ref[:, pl.ds(j * _CW, _CW)], -1.0, 1.0)
        return acc + _tree128(jnp.exp2(xc * _SE - _SE))

    acc = jax.lax.fori_loop(
        0, n_full, col_body, jnp.zeros((_R, 128), jnp.float32), unroll=4
    )
    s0 = jnp.sum(acc, axis=1)
    if tail:
        xc = jnp.clip(x_ref[:, pl.ds(n_full * _CW, tail)], -1.0, 1.0)
        s0 = s0 + jnp.sum(jnp.exp2(xc * _SE - _SE), axis=1)

    c = _pick_label(lane_ref, x128_ref, _R)
    row_loss = _margin_row_loss(s0, c, n_rows)

    @pl.when(i == 0)
    def _():
        out_ref[0, 0] = 0.0

    out_ref[0, 0] += jnp.sum(row_loss)


def _combine_body(a_ref, lane_ref, x128_ref, part_ref, xt_ref, out_ref, *,
                  n_rows, n_cols):
    i = pl.program_id(0)

    def cb(j, acc):
        return acc + part_ref[:, pl.ds(j * 128, 128)]

    acc = jax.lax.fori_loop(
        0, _SC_NJ, cb, jnp.zeros((_CR, 128), jnp.float32), unroll=4
    )
    s0 = jnp.sum(acc, axis=1)

    tail0 = _SC_NJ * _SC_CW
    valid = jax.lax.broadcasted_iota(jnp.int32, (_CR, _SC_CW), 1) < (n_cols - tail0)
    xc = jnp.clip(xt_ref[...], -1.0, 1.0)
    e = jnp.where(valid, jnp.exp2(xc * _SE - _SE), 0.0)
    s0 = s0 + jnp.sum(e, axis=1)

    c = _pick_label(lane_ref, x128_ref, _CR)
    row_loss = _margin_row_loss(s0, c, n_rows)

    @pl.when(i == 0)
    def _():
        out_ref[0, 0] = a_ref[0, 0]

    out_ref[0, 0] += jnp.sum(row_loss)


@jax.jit
def kernel(logits, labels):
    n_rows, n_cols = logits.shape
    labels = labels.astype(jnp.int32)
    tc_rows = n_rows - _SC_ROWS

    x128 = _sc_gather_rows(logits, labels)
    lane128 = (labels % 128).reshape(n_rows, 1)

    part = _sc_sumexp(logits)

    a = pl.pallas_call(
        functools.partial(_loss_body, n_rows=n_rows, n_cols=n_cols),
        grid=(tc_rows // _R,),
        in_specs=[
            pl.BlockSpec((_R, 1), lambda i: (i, 0)),
            pl.BlockSpec((_R, 128), lambda i: (i, 0)),
            pl.BlockSpec((_R, n_cols), lambda i: (i, 0)),
        ],
        out_specs=pl.BlockSpec((1, 1), lambda i: (0, 0), memory_space=pltpu.SMEM),
        out_shape=jax.ShapeDtypeStruct((1, 1), jnp.float32),
    )(lane128, x128, logits)

    blk0 = tc_rows // _CR
    out = pl.pallas_call(
        functools.partial(_combine_body, n_rows=n_rows, n_cols=n_cols),
        grid=(_SC_ROWS // _CR,),
        in_specs=[
            pl.BlockSpec((1, 1), lambda i: (0, 0), memory_space=pltpu.SMEM),
            pl.BlockSpec((_CR, 1), lambda i: (blk0 + i, 0)),
            pl.BlockSpec((_CR, 128), lambda i: (blk0 + i, 0)),
            pl.BlockSpec((_CR, _SC_NJ * 128), lambda i: (i, 0)),
            pl.BlockSpec((_CR, _SC_CW), lambda i: (blk0 + i, _SC_NJ)),
        ],
        out_specs=pl.BlockSpec((1, 1), lambda i: (0, 0), memory_space=pltpu.SMEM),
        out_shape=jax.ShapeDtypeStruct((1, 1), jnp.float32),
    )(a, lane128, x128, part, logits)
    return out[0, 0]


# speedup vs baseline: 3.4700x; 1.2714x over previous
import functools
import math

import jax
import jax.numpy as jnp
from jax.experimental import pallas as pl
from jax.experimental.pallas import tpu as pltpu
from jax.experimental.pallas import tpu_sc as plsc

_SCALE = 64.0
_MARGIN = 0.5
_COS_M = math.cos(_MARGIN)
_SIN_M = math.sin(_MARGIN)
_LOG2E = math.log2(math.e)
_SE = _SCALE * _LOG2E

_R = 32
_CW = 2048

_SC_ROWS = 256
_SC_CW = 1024
_SC_NJ = 97
_CR = 64


def _sc_gather_rows(logits, labels):
    n_rows, n_cols = logits.shape

    @functools.partial(
        pl.kernel,
        out_type=jax.ShapeDtypeStruct((n_rows, 128), logits.dtype),
        mesh=plsc.ScalarSubcoreMesh(axis_name="c", num_cores=2),
        scratch_types=[
            pltpu.SMEM((n_rows,), jnp.int32),
            pltpu.SemaphoreType.DMA,
            pltpu.SemaphoreType.DMA,
        ],
    )
    def gather_kernel(x_hbm, l_hbm, o_hbm, l_smem, sem_l, sem_d):
        core = jax.lax.axis_index("c")
        pltpu.async_copy(l_hbm, l_smem, sem_l).wait()
        half = n_rows // 2
        base = core * half

        @pl.loop(0, half)
        def _(i):
            r = base + i
            st = (l_smem[r] // 128) * 128
            pltpu.async_copy(x_hbm.at[r, pl.ds(st, 128)], o_hbm.at[r], sem_d)

        @pl.loop(0, half)
        def _(i):
            r = base + i
            st = (l_smem[r] // 128) * 128
            pltpu.make_async_copy(
                x_hbm.at[r, pl.ds(st, 128)], o_hbm.at[r], sem_d
            ).wait()

    return gather_kernel(logits, labels)


def _sc_sumexp(logits):
    n_rows, n_cols = logits.shape
    row_blk0 = (n_rows - _SC_ROWS) // 8

    @functools.partial(
        pl.kernel,
        out_type=jax.ShapeDtypeStruct((_SC_ROWS, _SC_NJ * 128), jnp.float32),
        mesh=plsc.VectorSubcoreMesh(core_axis_name="c", subcore_axis_name="s"),
    )
    def sumexp_kernel(x_hbm, o_hbm):
        def body(in_vmem, out_vmem):
            @pl.loop(0, 8)
            def _(r):
                for k in range(8):
                    acc = None
                    for m in range(8):
                        p = m * 8 + k
                        xc = jnp.clip(
                            in_vmem.at[pl.ds(r, 1), pl.ds(p * 16, 16)][...],
                            -1.0, 1.0,
                        )
                        e = jnp.exp(xc * _SCALE - _SCALE)
                        acc = e if acc is None else acc + e
                    out_vmem.at[pl.ds(r, 1), pl.ds(k * 16, 16)][...] = acc

        pltpu.emit_pipeline(
            body,
            grid=(_SC_ROWS // 8, _SC_NJ),
            in_specs=[
                pl.BlockSpec((8, _SC_CW), index_map=lambda i, j: (row_blk0 + i, j))
            ],
            out_specs=[pl.BlockSpec((8, 128), index_map=lambda i, j: (i, j))],
            core_axis_name=("c", "s"),
            dimension_semantics=(pltpu.PARALLEL, pltpu.PARALLEL),
        )(x_hbm, o_hbm)

    return sumexp_kernel(logits)


def _tree128(v):
    parts = [v[:, k * 128:(k + 1) * 128] for k in range(v.shape[1] // 128)]
    while len(parts) > 1:
        half = (len(parts) + 1) // 2
        parts = [
            parts[m] + parts[m + half] if m + half < len(parts) else parts[m]
            for m in range(half)
        ]
    return parts[0]


def _margin_row_loss(s0, c, n_rows):
    t_new = _SCALE * (c * _COS_M - _SIN_M * jnp.sqrt(jnp.maximum(1.0 - c * c, 0.0)))
    e_old = jnp.exp2(c * _SE - _SE)
    e_new = jnp.exp(t_new - _SCALE)
    s = s0 - e_old + e_new
    return (_SCALE + jnp.log(s) - t_new) * (1.0 / n_rows)


def _pick_label(lane_ref, x128_ref, rows):
    onehot = jax.lax.broadcasted_iota(jnp.int32, (rows, 128), 1) == lane_ref[...]
    return jnp.sum(jnp.where(onehot, jnp.clip(x128_ref[...], -1.0, 1.0), 0.0), axis=1)


def _loss_body(lane_ref, x128_ref, x_ref, out_ref, *, n_rows, n_cols):
    i = pl.program_id(0)

    n_full = n_cols // _CW
    tail = n_cols - n_full * _CW

    def col_body(j, acc):
        xc = jnp.clip(x_ref[:, pl.ds(j * _CW, _CW)], -1.0, 1.0)
        return acc + _tree128(jnp.exp2(xc * _SE - _SE))

    acc = jax.lax.fori_loop(
        0, n_full, col_body, jnp.zeros((_R, 128), jnp.float32), unroll=4
    )
    s0 = jnp.sum(acc, axis=1)
    if tail:
        xc = jnp.clip(x_ref[:, pl.ds(n_full * _CW, tail)], -1.0, 1.0)
        s0 = s0 + jnp.sum(jnp.exp2(xc * _SE - _SE), axis=1)

    c = _pick_label(lane_ref, x128_ref, _R)
    row_loss = _margin_row_loss(s0, c, n_rows)

    @pl.when(i == 0)
    def _():
        out_ref[0, 0] = 0.0

    out_ref[0, 0] += jnp.sum(row_loss)


def _combine_body(a_ref, lane_ref, x128_ref, part_ref, xt_ref, out_ref, *,
                  n_rows, n_cols):
    i = pl.program_id(0)

    def cb(j, acc):
        return acc + part_ref[:, pl.ds(j * 128, 128)]

    acc = jax.lax.fori_loop(
        0, _SC_NJ, cb, jnp.zeros((_CR, 128), jnp.float32), unroll=4
    )
    s0 = jnp.sum(acc, axis=1)

    tail0 = _SC_NJ * _SC_CW
    valid = jax.lax.broadcasted_iota(jnp.int32, (_CR, _SC_CW), 1) < (n_cols - tail0)
    xc = jnp.clip(xt_ref[...], -1.0, 1.0)
    e = jnp.where(valid, jnp.exp2(xc * _SE - _SE), 0.0)
    s0 = s0 + jnp.sum(e, axis=1)

    c = _pick_label(lane_ref, x128_ref, _CR)
    row_loss = _margin_row_loss(s0, c, n_rows)

    @pl.when(i == 0)
    def _():
        out_ref[0, 0] = a_ref[0, 0]

    out_ref[0, 0] += jnp.sum(row_loss)


@jax.jit
def kernel(logits, labels):
    n_rows, n_cols = logits.shape
    labels = labels.astype(jnp.int32)
    tc_rows = n_rows - _SC_ROWS

    x128 = _sc_gather_rows(logits, labels)
    lane128 = (labels % 128).reshape(n_rows, 1)

    part = _sc_sumexp(logits)

    a = pl.pallas_call(
        functools.partial(_loss_body, n_rows=n_rows, n_cols=n_cols),
        grid=(tc_rows // _R,),
        in_specs=[
            pl.BlockSpec((_R, 1), lambda i: (i, 0)),
            pl.BlockSpec((_R, 128), lambda i: (i, 0)),
            pl.BlockSpec((_R, n_cols), lambda i: (i, 0)),
        ],
        out_specs=pl.BlockSpec((1, 1), lambda i: (0, 0), memory_space=pltpu.SMEM),
        out_shape=jax.ShapeDtypeStruct((1, 1), jnp.float32),
    )(lane128, x128, logits)

    blk0 = tc_rows // _CR
    out = pl.pallas_call(
        functools.partial(_combine_body, n_rows=n_rows, n_cols=n_cols),
        grid=(_SC_ROWS // _CR,),
        in_specs=[
            pl.BlockSpec((1, 1), lambda i: (0, 0), memory_space=pltpu.SMEM),
            pl.BlockSpec((_CR, 1), lambda i: (blk0 + i, 0)),
            pl.BlockSpec((_CR, 128), lambda i: (blk0 + i, 0)),
            pl.BlockSpec((_CR, _SC_NJ * 128), lambda i: (i, 0)),
            pl.BlockSpec((_CR, _SC_CW), lambda i: (blk0 + i, _SC_NJ)),
        ],
        out_specs=pl.BlockSpec((1, 1), lambda i: (0, 0), memory_space=pltpu.SMEM),
        out_shape=jax.ShapeDtypeStruct((1, 1), jnp.float32),
    )(a, lane128, x128, part, logits)
    return out[0, 0]
